# baseline (speedup 1.0000x reference)
import jax
import jax.numpy as jnp
from jax import lax
from jax.experimental import pallas as pl
from jax.experimental.pallas import tpu as pltpu
from jax.experimental.pallas import tpu_sc as plsc

NC = 2
NS = 16
NW = NC * NS
L = 16


def _make_sc_kernel(n_points_q, n_frames_q, table_rows, table_cols):
    assert n_points_q % NW == 0
    b_per_w = n_points_q // NW
    chunk = 128
    n_chunks = b_per_w // chunk
    fgroups = (n_frames_q + L - 1) // L
    fpad = fgroups * L

    mesh = plsc.VectorSubcoreMesh(core_axis_name="c", subcore_axis_name="s")

    def body(frames_hbm, points_hbm, table_hbm, out_hbm,
             frames_v, idx_v, rows_v, out_v, sem):
        c = lax.axis_index("c")
        s = lax.axis_index("s")
        wid = s * NC + c
        row0 = wid * b_per_w

        pltpu.sync_copy(frames_hbm, frames_v)
        f_regs = [frames_v[pl.ds(g * L, L)] for g in range(fgroups)]

        for ch in range(n_chunks):
            base = row0 + ch * chunk
            pltpu.sync_copy(points_hbm.at[pl.ds(base, chunk)], idx_v)
            pltpu.async_copy(table_hbm.at[idx_v], rows_v, sem).wait()

            def row_body(r, carry):
                rvec = jnp.full((L,), r, dtype=jnp.int32)
                for g in range(fgroups):
                    vals = plsc.load_gather(rows_v, [rvec, f_regs[g]])
                    res = jnp.where(vals > 0.0, vals + 1.0, jnp.exp(vals))
                    out_v[pl.ds(r * n_frames_q + g * L, L)] = res
                return carry

            lax.fori_loop(0, chunk, row_body, 0)

            out_words = chunk * n_frames_q
            pltpu.sync_copy(out_v.at[pl.ds(0, out_words)],
                            out_hbm.at[pl.ds(base * n_frames_q, out_words)])

    kern = pl.kernel(
        body,
        out_type=jax.ShapeDtypeStruct((n_points_q * n_frames_q,), jnp.float32),
        mesh=mesh,
        scratch_types=[
            pltpu.VMEM((fpad,), jnp.int32),
            pltpu.VMEM((chunk,), jnp.int32),
            pltpu.VMEM((chunk, table_cols), jnp.float32),
            pltpu.VMEM((chunk * n_frames_q + L,), jnp.float32),
            pltpu.SemaphoreType.DMA,
        ],
        compiler_params=pltpu.CompilerParams(use_tc_tiling_on_sc=False,
                                             needs_layout_passes=False),
    )
    return kern, fpad


def kernel(frames, points, uncertainty):
    n_rows, n_cols = uncertainty.shape[0], uncertainty.shape[1]
    p_q = points.shape[0]
    f_q = frames.shape[0]
    table = uncertainty.reshape(n_rows, n_cols)
    kern, fpad = _make_sc_kernel(p_q, f_q, n_rows, n_cols)
    frames_pad = jnp.concatenate(
        [frames.astype(jnp.int32),
         jnp.zeros((fpad - f_q,), dtype=jnp.int32)])
    out = kern(frames_pad, points.astype(jnp.int32), table)
    return out.reshape(p_q, f_q, 1)


# speedup vs baseline: 1.1529x; 1.1529x over previous
import jax
import jax.numpy as jnp
from jax import lax
from jax.experimental import pallas as pl
from jax.experimental.pallas import tpu as pltpu
from jax.experimental.pallas import tpu_sc as plsc

NC = 2
NS = 16
NW = NC * NS
L = 16


def _make_sc_kernel(n_points_q, n_frames_q, table_rows, table_cols):
    assert n_points_q % NW == 0
    b_per_w = n_points_q // NW
    chunk = 128
    n_chunks = b_per_w // chunk
    fgroups = (n_frames_q + L - 1) // L
    fpad = fgroups * L

    mesh = plsc.VectorSubcoreMesh(core_axis_name="c", subcore_axis_name="s")

    def body(frames_hbm, points_hbm, table_hbm, out_hbm,
             frames_v, idx_v, rows_v, out_v, sem):
        c = lax.axis_index("c")
        s = lax.axis_index("s")
        wid = s * NC + c
        row0 = wid * b_per_w

        pltpu.sync_copy(frames_hbm, frames_v)
        f_regs = [frames_v[pl.ds(g * L, L)] for g in range(fgroups)]
        zvec = jnp.zeros((L,), dtype=jnp.int32)

        for ch in range(n_chunks):
            base = row0 + ch * chunk
            pltpu.sync_copy(points_hbm.at[pl.ds(base, chunk)], idx_v)
            pltpu.async_copy(table_hbm.at[idx_v], rows_v, sem).wait()

            def row_body(r, carry):
                rvec = jnp.full((L,), r, dtype=jnp.int32)
                for g in range(fgroups):
                    vals = plsc.load_gather(rows_v, [rvec, f_regs[g]])
                    res = jnp.where(vals > 0.0, vals + 1.0, jnp.exp(vals))
                    out_v[pl.ds(r * n_frames_q + g * L, L)] = res
                return carry

            lax.fori_loop(0, chunk, row_body, 0)

            out_words = chunk * n_frames_q
            pltpu.sync_copy(out_v.at[pl.ds(0, out_words)],
                            out_hbm.at[pl.ds(base * n_frames_q, out_words)])

    kern = pl.kernel(
        body,
        out_type=jax.ShapeDtypeStruct((n_points_q * n_frames_q,), jnp.float32),
        mesh=mesh,
        scratch_types=[
            pltpu.VMEM((fpad,), jnp.int32),
            pltpu.VMEM((chunk,), jnp.int32),
            pltpu.VMEM((chunk, table_cols), jnp.float32),
            pltpu.VMEM((chunk * n_frames_q + L,), jnp.float32),
            pltpu.SemaphoreType.DMA,
        ],
        compiler_params=pltpu.CompilerParams(needs_layout_passes=False),
    )
    return kern, fpad


def kernel(frames, points, uncertainty):
    n_rows, n_cols = uncertainty.shape[0], uncertainty.shape[1]
    p_q = points.shape[0]
    f_q = frames.shape[0]
    table = uncertainty.reshape(n_rows, n_cols)
    cols_pad = (-n_cols) % 128
    if cols_pad:
        table = jnp.pad(table, ((0, 0), (0, cols_pad)))
    kern, fpad = _make_sc_kernel(p_q, f_q, n_rows, n_cols + cols_pad)
    frames_pad = jnp.concatenate(
        [frames.astype(jnp.int32),
         jnp.zeros((fpad - f_q,), dtype=jnp.int32)])
    out = kern(frames_pad, points.astype(jnp.int32), table)
    return out.reshape(p_q, f_q, 1)
